# Initial kernel scaffold; baseline (speedup 1.0000x reference)
import jax
import jax.numpy as jnp
from jax.experimental import pallas as pl


def kernel(node_feature, node_type, edge_time, edge_index, edge_type, Wa, ba, Qw, Qb, Kw, Kb, Vw, Vb, Aw, Ab, rel_pri, rel_att, rel_msg, skip, Wc, bc):
    raise NotImplementedError("write your pallas kernel here")


# speedup vs baseline: 39.9136x; 39.9136x over previous
import functools

import jax
import jax.numpy as jnp
from jax import lax
from jax.experimental import pallas as pl
from jax.experimental.pallas import tpu as pltpu
from jax.experimental.pallas import tpu_sc as plsc

N = 50000
E = 800000
D_IN = 128
D = 64
T = 2
R = 4
H = 4
DK = D // H
L = 2
OUT = 2

NC = 2
NS = 16
LANES = 16

EPT = E // NS
EC = 48
NCHUNK = EPT // EC
ECT = EPT - NCHUNK * EC
ACC_W = 34
RSTR = 3200
ZB_FULL = 66
ZB_LAST = 41

_BLK = 2000
_GRID = N // _BLK


def _adapt_body(x_ref, nt_ref, wa_ref, ba_ref, h_ref):
    x = x_ref[...]
    y0 = jnp.dot(x, wa_ref[0], preferred_element_type=jnp.float32) + ba_ref[0, :][None, :]
    y1 = jnp.dot(x, wa_ref[1], preferred_element_type=jnp.float32) + ba_ref[1, :][None, :]
    sel = (nt_ref[...] == 0)
    h_ref[...] = jnp.tanh(jnp.where(sel, y0, y1))


def _adapt(x, nt, Wa, ba):
    return pl.pallas_call(
        _adapt_body,
        grid=(_GRID,),
        in_specs=[
            pl.BlockSpec((_BLK, D_IN), lambda i: (i, 0)),
            pl.BlockSpec((_BLK, 1), lambda i: (i, 0)),
            pl.BlockSpec((T, D_IN, D), lambda i: (0, 0, 0)),
            pl.BlockSpec((T, D), lambda i: (0, 0)),
        ],
        out_specs=pl.BlockSpec((_BLK, D), lambda i: (i, 0)),
        out_shape=jax.ShapeDtypeStruct((N, D), jnp.float32),
    )(x, nt, Wa, ba)


def _typed(h, nt0, w_ref, b_ref):
    y0 = jnp.dot(h, w_ref[0], preferred_element_type=jnp.float32) + b_ref[0, :][None, :]
    y1 = jnp.dot(h, w_ref[1], preferred_element_type=jnp.float32) + b_ref[1, :][None, :]
    return jnp.where(nt0, y0, y1)


def _tables_body(h_ref, nt_ref, qw_ref, qb_ref, kw_ref, kb_ref, vw_ref, vb_ref,
                 wat_ref, wms_ref, tab_ref, qp_ref):
    h = h_ref[...]
    nt0 = (nt_ref[...] == 0)
    q = _typed(h, nt0, qw_ref, qb_ref)
    k = _typed(h, nt0, kw_ref, kb_ref)
    v = _typed(h, nt0, vw_ref, vb_ref)
    qp_ref[...] = jnp.concatenate([q, jnp.zeros_like(q)], axis=1)
    for r in range(R):
        kt = jnp.dot(k, wat_ref[r], preferred_element_type=jnp.float32)
        mt = jnp.dot(v, wms_ref[r], preferred_element_type=jnp.float32)
        tab_ref[r] = jnp.concatenate([kt, mt], axis=1)


def _tables(h, nt, qw, qb, kw, kb, vw, vb, wat, wms):
    wspec = lambda shp: pl.BlockSpec(shp, lambda i: tuple(0 for _ in shp))
    return pl.pallas_call(
        _tables_body,
        grid=(_GRID,),
        in_specs=[
            pl.BlockSpec((_BLK, D), lambda i: (i, 0)),
            pl.BlockSpec((_BLK, 1), lambda i: (i, 0)),
            wspec((T, D, D)), wspec((T, D)),
            wspec((T, D, D)), wspec((T, D)),
            wspec((T, D, D)), wspec((T, D)),
            wspec((R, D, D)), wspec((R, D, D)),
        ],
        out_specs=[
            pl.BlockSpec((R, _BLK, 2 * D), lambda i: (0, i, 0)),
            pl.BlockSpec((_BLK, 2 * D), lambda i: (i, 0)),
        ],
        out_shape=[
            jax.ShapeDtypeStruct((R, N, 2 * D), jnp.float32),
            jax.ShapeDtypeStruct((N, 2 * D), jnp.float32),
        ],
    )(h, nt, qw, qb, kw, kb, vw, vb, wat, wms)


def _update_body(o0_ref, o1_ref, h_ref, nt_ref, aw_ref, ab_ref, al_ref, ho_ref):
    eps = 1e-16
    o0 = o0_ref[...]
    o1 = o1_ref[...]
    parts = []
    for src, base in ((o0, 0), (o1, 0)):
        for hh in range(2):
            den = src[:, 32 + hh:33 + hh]
            parts.append(src[:, hh * 16:(hh + 1) * 16] / (den + eps))
    agg = jnp.concatenate(parts, axis=1)
    g = agg * 0.5 * (1.0 + lax.erf(agg * (2.0 ** -0.5)))
    nt0 = (nt_ref[...] == 0)
    trans = _typed(g, nt0, aw_ref, ab_ref)
    alpha = jnp.where(nt0, al_ref[0, 0], al_ref[0, 1])
    ho_ref[...] = alpha * trans + (1.0 - alpha) * h_ref[...]


def _update(o0, o1, h, nt, aw, ab, alpha):
    return pl.pallas_call(
        _update_body,
        grid=(_GRID,),
        in_specs=[
            pl.BlockSpec((_BLK, ACC_W), lambda i: (i, 0)),
            pl.BlockSpec((_BLK, ACC_W), lambda i: (i, 0)),
            pl.BlockSpec((_BLK, D), lambda i: (i, 0)),
            pl.BlockSpec((_BLK, 1), lambda i: (i, 0)),
            pl.BlockSpec((T, D, D), lambda i: (0, 0, 0)),
            pl.BlockSpec((T, D), lambda i: (0, 0)),
            pl.BlockSpec((1, T), lambda i: (0, 0)),
        ],
        out_specs=pl.BlockSpec((_BLK, D), lambda i: (i, 0)),
        out_shape=jax.ShapeDtypeStruct((N, D), jnp.float32),
    )(o0, o1, h, nt, aw, ab, alpha)


def _head_body(h_ref, wc_ref, bc_ref, out_ref):
    out_ref[...] = (jnp.dot(h_ref[...], wc_ref[...],
                            preferred_element_type=jnp.float32)
                    + bc_ref[0, :][None, :])


def _head(h, Wc, bc):
    return pl.pallas_call(
        _head_body,
        grid=(_GRID,),
        in_specs=[
            pl.BlockSpec((_BLK, D), lambda i: (i, 0)),
            pl.BlockSpec((D, OUT), lambda i: (0, 0)),
            pl.BlockSpec((1, OUT), lambda i: (0, 0)),
        ],
        out_specs=pl.BlockSpec((_BLK, OUT), lambda i: (i, 0)),
        out_shape=jax.ShapeDtypeStruct((N, OUT), jnp.float32),
    )(h, Wc, bc)


def _edge_chunks(tab, qtab, sidx_hbm, dst_hbm, sid, acc, srcb, dstb,
                 srct, dstt, tabb, qb, msgb, sem_t, sem_q, ko, mo, qo):
    iota = lax.iota(jnp.int32, LANES)
    zero16 = jnp.zeros((LANES,), jnp.float32)
    perms = [jnp.bitwise_xor(iota, sh) for sh in (8, 4, 2, 1)]

    def allsum(v):
        for p in perms:
            v = v + v[p]
        return v

    ebase = sid * EPT

    def do_chunk(base, n, sb, db):
        pltpu.sync_copy(sidx_hbm.at[pl.ds(base, n)], sb)
        pltpu.sync_copy(dst_hbm.at[pl.ds(base, n)], db)

        cp_t = pltpu.async_copy(tab.at[sb], tabb.at[pl.ds(0, n)], sem_t)
        cp_q = pltpu.async_copy(qtab.at[db], qb.at[pl.ds(0, n)], sem_q)
        cp_t.wait()
        cp_q.wait()

        def edge(e, _):
            q0 = qb[e, pl.ds(qo, 16)]
            q1 = qb[e, pl.ds(qo + 16, 16)]
            k0 = tabb[e, pl.ds(ko, 16)]
            k1 = tabb[e, pl.ds(ko + 16, 16)]
            m0 = tabb[e, pl.ds(mo, 16)]
            m1 = tabb[e, pl.ds(mo + 16, 16)]
            w0 = jnp.exp(allsum(k0 * q0))
            w1 = jnp.exp(allsum(k1 * q1))
            den = jnp.where(iota == 14, w0, jnp.where(iota == 15, w1, zero16))
            msgb[e, pl.ds(18, 16)] = den
            msgb[e, pl.ds(0, 16)] = m0 * w0
            msgb[e, pl.ds(16, 16)] = m1 * w1
            return 0
        lax.fori_loop(0, n, edge, 0)

        pltpu.sync_copy(msgb.at[pl.ds(0, n)], acc.at[db], add=True)

    def chunk(c, _):
        do_chunk(ebase + c * EC, EC, srcb, dstb)
        return 0

    lax.fori_loop(0, NCHUNK, chunk, 0)
    do_chunk(ebase + NCHUNK * EC, ECT, srct, dstt)


def _edge_kernel_body(tab, qp, sidx, dst, out0, out1,
                      acc, srcb, dstb, srct, dstt,
                      tabb, qb, msgb, sem_t, sem_q):
    cid = lax.axis_index("c")
    sid = lax.axis_index("s")
    zero16 = jnp.zeros((LANES,), jnp.float32)
    iota = lax.iota(jnp.int32, LANES)

    def fill_idx(buf, nwords, base):
        def st(g, _):
            buf[pl.ds(g * 16, 16)] = iota + (base + g * 16)
            return 0
        lax.fori_loop(0, nwords // 16, st, 0)

    def zrow(e, _):
        msgb[e, pl.ds(0, 16)] = zero16
        msgb[e, pl.ds(16, 16)] = zero16
        msgb[e, pl.ds(18, 16)] = zero16
        return 0
    lax.fori_loop(0, EC, zrow, 0)
    zbase = sid * RSTR
    nzb = jnp.where(sid == NS - 1, ZB_LAST, ZB_FULL)

    def zcopy(zb, _):
        fill_idx(srcb, EC, zbase + zb * EC)
        pltpu.sync_copy(msgb, acc.at[srcb])
        return 0
    lax.fori_loop(0, nzb, zcopy, 0)
    fill_idx(srct, ECT, zbase + nzb * EC)
    pltpu.sync_copy(msgb.at[pl.ds(0, ECT)], acc.at[srct])
    plsc.subcore_barrier()

    @pl.when(cid == 0)
    def _():
        _edge_chunks(tab, qp, sidx, dst, sid, acc, srcb, dstb, srct, dstt,
                     tabb, qb, msgb, sem_t, sem_q, 0, 64, 0)

    @pl.when(cid == 1)
    def _():
        _edge_chunks(tab, qp, sidx, dst, sid, acc, srcb, dstb, srct, dstt,
                     tabb, qb, msgb, sem_t, sem_q, 32, 96, 32)

    plsc.subcore_barrier()

    def wcopy_to(out):
        def wcopy(zb, _):
            fill_idx(srcb, EC, zbase + zb * EC)
            pltpu.sync_copy(acc.at[srcb], msgb)
            pltpu.sync_copy(msgb, out.at[pl.ds(zbase + zb * EC, EC)])
            return 0
        return wcopy

    def wtail_to(out):
        fill_idx(srct, ECT, zbase + nzb * EC)
        pltpu.sync_copy(acc.at[srct], msgb.at[pl.ds(0, ECT)])
        pltpu.sync_copy(msgb.at[pl.ds(0, ECT)],
                        out.at[pl.ds(zbase + nzb * EC, ECT)])

    @pl.when(cid == 0)
    def _():
        lax.fori_loop(0, nzb, wcopy_to(out0), 0)
        wtail_to(out0)

    @pl.when(cid == 1)
    def _():
        lax.fori_loop(0, nzb, wcopy_to(out1), 0)
        wtail_to(out1)


@functools.lru_cache(maxsize=1)
def _get_edge_kernel():
    return pl.kernel(
        _edge_kernel_body,
        out_type=[
            jax.ShapeDtypeStruct((N, ACC_W), jnp.float32),
            jax.ShapeDtypeStruct((N, ACC_W), jnp.float32),
        ],
        mesh=plsc.VectorSubcoreMesh(core_axis_name="c", subcore_axis_name="s",
                                    num_cores=NC, num_subcores=NS),
        scratch_types=[
            pltpu.VMEM_SHARED((N, ACC_W), jnp.float32),
            pltpu.VMEM((EC,), jnp.int32),
            pltpu.VMEM((EC,), jnp.int32),
            pltpu.VMEM((ECT,), jnp.int32),
            pltpu.VMEM((ECT,), jnp.int32),
            pltpu.VMEM((EC, 2 * D), jnp.float32),
            pltpu.VMEM((EC, 2 * D), jnp.float32),
            pltpu.VMEM((EC, ACC_W), jnp.float32),
            pltpu.SemaphoreType.DMA,
            pltpu.SemaphoreType.DMA,
        ],
    )


def _sidx_body(src_ref, typ_ref, out_ref):
    out_ref[...] = typ_ref[...] * N + src_ref[...]


def _sidx(src, typ):
    e2 = E // 128
    blk = e2
    return pl.pallas_call(
        _sidx_body,
        grid=(e2 // blk,),
        in_specs=[
            pl.BlockSpec((blk, 128), lambda i: (i, 0)),
            pl.BlockSpec((blk, 128), lambda i: (i, 0)),
        ],
        out_specs=pl.BlockSpec((blk, 128), lambda i: (i, 0)),
        out_shape=jax.ShapeDtypeStruct((e2, 128), jnp.int32),
    )(src.reshape(e2, 128), typ.reshape(e2, 128)).reshape(E)


def _blockdiag(mats):
    r, h, dk, _ = mats.shape
    eye = jnp.eye(h, dtype=mats.dtype)
    big = jnp.einsum('rhij,hg->rhigj', mats, eye).reshape(r, h, dk, h * dk)
    return big.reshape(r, h * dk, h * dk)


def kernel(node_feature, node_type, edge_time, edge_index, edge_type,
           Wa, ba, Qw, Qb, Kw, Kb, Vw, Vb, Aw, Ab,
           rel_pri, rel_att, rel_msg, skip, Wc, bc):
    del edge_time
    nt = node_type.astype(jnp.int32).reshape(N, 1)
    src = edge_index[0].astype(jnp.int32)
    dst = edge_index[1].astype(jnp.int32)
    typ = edge_type.astype(jnp.int32)
    sidx = _sidx(src, typ)

    h = _adapt(node_feature, nt, Wa, ba)

    for l in range(L):
        att_scaled = rel_att[l] * (rel_pri[l] / float(DK) ** 0.5)[:, :, None, None]
        wat = _blockdiag(att_scaled)
        wms = _blockdiag(rel_msg[l])
        tab, qp = _tables(
            h, nt, Qw[l], Qb[l], Kw[l], Kb[l], Vw[l], Vb[l], wat, wms)
        o0, o1 = _get_edge_kernel()(
            tab.reshape(R * N, 2 * D), qp, sidx, dst)
        alpha = jax.nn.sigmoid(skip[l]).reshape(1, T)
        h = _update(o0, o1, h, nt, Aw[l], Ab[l], alpha)

    return _head(h, Wc, bc.reshape(1, OUT))


# speedup vs baseline: 47.3235x; 1.1856x over previous
import functools

import jax
import jax.numpy as jnp
from jax import lax
from jax.experimental import pallas as pl
from jax.experimental.pallas import tpu as pltpu
from jax.experimental.pallas import tpu_sc as plsc

N = 50000
E = 800000
D_IN = 128
D = 64
T = 2
R = 4
H = 4
DK = D // H
L = 2
OUT = 2

NC = 2
NS = 16
LANES = 16

EPT = E // NS
EC = 48
NCHUNK = EPT // EC
ECT = EPT - NCHUNK * EC
ACC_W = 34
RSTR = 3200
ZB_FULL = 66
ZB_LAST = 41

_BLK = 2000
_GRID = N // _BLK


def _adapt_body(x_ref, nt_ref, wa_ref, ba_ref, h_ref):
    x = x_ref[...]
    y0 = jnp.dot(x, wa_ref[0], preferred_element_type=jnp.float32) + ba_ref[0, :][None, :]
    y1 = jnp.dot(x, wa_ref[1], preferred_element_type=jnp.float32) + ba_ref[1, :][None, :]
    sel = (nt_ref[...] == 0)
    h_ref[...] = jnp.tanh(jnp.where(sel, y0, y1))


def _adapt(x, nt, Wa, ba):
    return pl.pallas_call(
        _adapt_body,
        grid=(_GRID,),
        in_specs=[
            pl.BlockSpec((_BLK, D_IN), lambda i: (i, 0)),
            pl.BlockSpec((_BLK, 1), lambda i: (i, 0)),
            pl.BlockSpec((T, D_IN, D), lambda i: (0, 0, 0)),
            pl.BlockSpec((T, D), lambda i: (0, 0)),
        ],
        out_specs=pl.BlockSpec((_BLK, D), lambda i: (i, 0)),
        out_shape=jax.ShapeDtypeStruct((N, D), jnp.float32),
    )(x, nt, Wa, ba)


def _typed(h, nt0, w_ref, b_ref):
    y0 = jnp.dot(h, w_ref[0], preferred_element_type=jnp.float32) + b_ref[0, :][None, :]
    y1 = jnp.dot(h, w_ref[1], preferred_element_type=jnp.float32) + b_ref[1, :][None, :]
    return jnp.where(nt0, y0, y1)


def _tables_body(h_ref, nt_ref, qw_ref, qb_ref, kw_ref, kb_ref, vw_ref, vb_ref,
                 wat_ref, wms_ref, tab_ref, qp_ref):
    h = h_ref[...]
    nt0 = (nt_ref[...] == 0)
    q = _typed(h, nt0, qw_ref, qb_ref)
    k = _typed(h, nt0, kw_ref, kb_ref)
    v = _typed(h, nt0, vw_ref, vb_ref)
    qp_ref[...] = jnp.concatenate([q, jnp.zeros_like(q)], axis=1)
    for r in range(R):
        kt = jnp.dot(k, wat_ref[r], preferred_element_type=jnp.float32)
        mt = jnp.dot(v, wms_ref[r], preferred_element_type=jnp.float32)
        tab_ref[r] = jnp.concatenate([kt, mt], axis=1)


def _tables(h, nt, qw, qb, kw, kb, vw, vb, wat, wms):
    wspec = lambda shp: pl.BlockSpec(shp, lambda i: tuple(0 for _ in shp))
    return pl.pallas_call(
        _tables_body,
        grid=(_GRID,),
        in_specs=[
            pl.BlockSpec((_BLK, D), lambda i: (i, 0)),
            pl.BlockSpec((_BLK, 1), lambda i: (i, 0)),
            wspec((T, D, D)), wspec((T, D)),
            wspec((T, D, D)), wspec((T, D)),
            wspec((T, D, D)), wspec((T, D)),
            wspec((R, D, D)), wspec((R, D, D)),
        ],
        out_specs=[
            pl.BlockSpec((R, _BLK, 2 * D), lambda i: (0, i, 0)),
            pl.BlockSpec((_BLK, 2 * D), lambda i: (i, 0)),
        ],
        out_shape=[
            jax.ShapeDtypeStruct((R, N, 2 * D), jnp.float32),
            jax.ShapeDtypeStruct((N, 2 * D), jnp.float32),
        ],
    )(h, nt, qw, qb, kw, kb, vw, vb, wat, wms)


def _update_body(o0_ref, o1_ref, h_ref, nt_ref, aw_ref, ab_ref, al_ref, ho_ref):
    eps = 1e-16
    o0 = o0_ref[...]
    o1 = o1_ref[...]
    parts = []
    for src, base in ((o0, 0), (o1, 0)):
        for hh in range(2):
            den = src[:, 32 + hh:33 + hh]
            parts.append(src[:, hh * 16:(hh + 1) * 16] / (den + eps))
    agg = jnp.concatenate(parts, axis=1)
    g = agg * 0.5 * (1.0 + lax.erf(agg * (2.0 ** -0.5)))
    nt0 = (nt_ref[...] == 0)
    trans = _typed(g, nt0, aw_ref, ab_ref)
    alpha = jnp.where(nt0, al_ref[0, 0], al_ref[0, 1])
    ho_ref[...] = alpha * trans + (1.0 - alpha) * h_ref[...]


def _update(o0, o1, h, nt, aw, ab, alpha):
    return pl.pallas_call(
        _update_body,
        grid=(_GRID,),
        in_specs=[
            pl.BlockSpec((_BLK, ACC_W), lambda i: (i, 0)),
            pl.BlockSpec((_BLK, ACC_W), lambda i: (i, 0)),
            pl.BlockSpec((_BLK, D), lambda i: (i, 0)),
            pl.BlockSpec((_BLK, 1), lambda i: (i, 0)),
            pl.BlockSpec((T, D, D), lambda i: (0, 0, 0)),
            pl.BlockSpec((T, D), lambda i: (0, 0)),
            pl.BlockSpec((1, T), lambda i: (0, 0)),
        ],
        out_specs=pl.BlockSpec((_BLK, D), lambda i: (i, 0)),
        out_shape=jax.ShapeDtypeStruct((N, D), jnp.float32),
    )(o0, o1, h, nt, aw, ab, alpha)


def _head_body(h_ref, wc_ref, bc_ref, out_ref):
    out_ref[...] = (jnp.dot(h_ref[...], wc_ref[...],
                            preferred_element_type=jnp.float32)
                    + bc_ref[0, :][None, :])


def _head(h, Wc, bc):
    return pl.pallas_call(
        _head_body,
        grid=(_GRID,),
        in_specs=[
            pl.BlockSpec((_BLK, D), lambda i: (i, 0)),
            pl.BlockSpec((D, OUT), lambda i: (0, 0)),
            pl.BlockSpec((1, OUT), lambda i: (0, 0)),
        ],
        out_specs=pl.BlockSpec((_BLK, OUT), lambda i: (i, 0)),
        out_shape=jax.ShapeDtypeStruct((N, OUT), jnp.float32),
    )(h, Wc, bc)


def _edge_chunks(tab, qtab, sidx_hbm, dst_hbm, sid, acc, srcb, dstb,
                 srct, dstt, tabb, qb, msgb, sem_t, sem_q, ko, mo, qo):
    iota = lax.iota(jnp.int32, LANES)
    zero16 = jnp.zeros((LANES,), jnp.float32)
    perms = [jnp.bitwise_xor(iota, sh) for sh in (8, 4, 2, 1)]

    def allsum(v):
        for p in perms:
            v = v + v[p]
        return v

    ebase = sid * EPT

    def do_chunk(base, n, sb, db):
        ci_s = pltpu.async_copy(sidx_hbm.at[pl.ds(base, n)], sb, sem_t)
        ci_d = pltpu.async_copy(dst_hbm.at[pl.ds(base, n)], db, sem_q)
        ci_s.wait()
        ci_d.wait()

        cp_t = pltpu.async_copy(tab.at[sb], tabb.at[pl.ds(0, n)], sem_t)
        cp_q = pltpu.async_copy(qtab.at[db], qb.at[pl.ds(0, n)], sem_q)
        cp_t.wait()
        cp_q.wait()

        def edge(e, _):
            q0 = qb[e, pl.ds(qo, 16)]
            q1 = qb[e, pl.ds(qo + 16, 16)]
            k0 = tabb[e, pl.ds(ko, 16)]
            k1 = tabb[e, pl.ds(ko + 16, 16)]
            m0 = tabb[e, pl.ds(mo, 16)]
            m1 = tabb[e, pl.ds(mo + 16, 16)]
            w0 = jnp.exp(allsum(k0 * q0))
            w1 = jnp.exp(allsum(k1 * q1))
            den = jnp.where(iota == 14, w0, jnp.where(iota == 15, w1, zero16))
            msgb[e, pl.ds(18, 16)] = den
            msgb[e, pl.ds(0, 16)] = m0 * w0
            msgb[e, pl.ds(16, 16)] = m1 * w1
            return 0
        lax.fori_loop(0, n, edge, 0)

        pltpu.sync_copy(msgb.at[pl.ds(0, n)], acc.at[db], add=True)

    def chunk(c, _):
        do_chunk(ebase + c * EC, EC, srcb, dstb)
        return 0

    lax.fori_loop(0, NCHUNK, chunk, 0)
    do_chunk(ebase + NCHUNK * EC, ECT, srct, dstt)


def _edge_kernel_body(tab, qp, sidx, dst, out0, out1,
                      acc, srcb, dstb, srct, dstt,
                      tabb, qb, msgb, sem_t, sem_q):
    cid = lax.axis_index("c")
    sid = lax.axis_index("s")
    zero16 = jnp.zeros((LANES,), jnp.float32)
    iota = lax.iota(jnp.int32, LANES)

    def fill_idx(buf, nwords, base):
        def st(g, _):
            buf[pl.ds(g * 16, 16)] = iota + (base + g * 16)
            return 0
        lax.fori_loop(0, nwords // 16, st, 0)

    def zrow(e, _):
        msgb[e, pl.ds(0, 16)] = zero16
        msgb[e, pl.ds(16, 16)] = zero16
        msgb[e, pl.ds(18, 16)] = zero16
        return 0
    lax.fori_loop(0, EC, zrow, 0)
    zbase = sid * RSTR
    nzb = jnp.where(sid == NS - 1, ZB_LAST, ZB_FULL)

    def zcopy(zb, _):
        fill_idx(srcb, EC, zbase + zb * EC)
        pltpu.sync_copy(msgb, acc.at[srcb])
        return 0
    lax.fori_loop(0, nzb, zcopy, 0)
    fill_idx(srct, ECT, zbase + nzb * EC)
    pltpu.sync_copy(msgb.at[pl.ds(0, ECT)], acc.at[srct])
    plsc.subcore_barrier()

    @pl.when(cid == 0)
    def _():
        _edge_chunks(tab, qp, sidx, dst, sid, acc, srcb, dstb, srct, dstt,
                     tabb, qb, msgb, sem_t, sem_q, 0, 64, 0)

    @pl.when(cid == 1)
    def _():
        _edge_chunks(tab, qp, sidx, dst, sid, acc, srcb, dstb, srct, dstt,
                     tabb, qb, msgb, sem_t, sem_q, 32, 96, 32)

    plsc.subcore_barrier()

    def wcopy_to(out):
        def wcopy(zb, _):
            fill_idx(srcb, EC, zbase + zb * EC)
            pltpu.sync_copy(acc.at[srcb], msgb)
            pltpu.sync_copy(msgb, out.at[pl.ds(zbase + zb * EC, EC)])
            return 0
        return wcopy

    def wtail_to(out):
        fill_idx(srct, ECT, zbase + nzb * EC)
        pltpu.sync_copy(acc.at[srct], msgb.at[pl.ds(0, ECT)])
        pltpu.sync_copy(msgb.at[pl.ds(0, ECT)],
                        out.at[pl.ds(zbase + nzb * EC, ECT)])

    @pl.when(cid == 0)
    def _():
        lax.fori_loop(0, nzb, wcopy_to(out0), 0)
        wtail_to(out0)

    @pl.when(cid == 1)
    def _():
        lax.fori_loop(0, nzb, wcopy_to(out1), 0)
        wtail_to(out1)


@functools.lru_cache(maxsize=1)
def _get_edge_kernel():
    return pl.kernel(
        _edge_kernel_body,
        out_type=[
            jax.ShapeDtypeStruct((N, ACC_W), jnp.float32),
            jax.ShapeDtypeStruct((N, ACC_W), jnp.float32),
        ],
        mesh=plsc.VectorSubcoreMesh(core_axis_name="c", subcore_axis_name="s",
                                    num_cores=NC, num_subcores=NS),
        scratch_types=[
            pltpu.VMEM_SHARED((N, ACC_W), jnp.float32),
            pltpu.VMEM((EC,), jnp.int32),
            pltpu.VMEM((EC,), jnp.int32),
            pltpu.VMEM((ECT,), jnp.int32),
            pltpu.VMEM((ECT,), jnp.int32),
            pltpu.VMEM((EC, 2 * D), jnp.float32),
            pltpu.VMEM((EC, 2 * D), jnp.float32),
            pltpu.VMEM((EC, ACC_W), jnp.float32),
            pltpu.SemaphoreType.DMA,
            pltpu.SemaphoreType.DMA,
        ],
    )


def _sidx_body(src_ref, typ_ref, out_ref):
    out_ref[...] = typ_ref[...] * N + src_ref[...]


def _sidx(src, typ):
    e2 = E // 128
    blk = e2
    return pl.pallas_call(
        _sidx_body,
        grid=(e2 // blk,),
        in_specs=[
            pl.BlockSpec((blk, 128), lambda i: (i, 0)),
            pl.BlockSpec((blk, 128), lambda i: (i, 0)),
        ],
        out_specs=pl.BlockSpec((blk, 128), lambda i: (i, 0)),
        out_shape=jax.ShapeDtypeStruct((e2, 128), jnp.int32),
    )(src.reshape(e2, 128), typ.reshape(e2, 128)).reshape(E)


def _blockdiag(mats):
    r, h, dk, _ = mats.shape
    eye = jnp.eye(h, dtype=mats.dtype)
    big = jnp.einsum('rhij,hg->rhigj', mats, eye).reshape(r, h, dk, h * dk)
    return big.reshape(r, h * dk, h * dk)


def kernel(node_feature, node_type, edge_time, edge_index, edge_type,
           Wa, ba, Qw, Qb, Kw, Kb, Vw, Vb, Aw, Ab,
           rel_pri, rel_att, rel_msg, skip, Wc, bc):
    del edge_time
    nt = node_type.astype(jnp.int32).reshape(N, 1)
    src = edge_index[0].astype(jnp.int32)
    dst = edge_index[1].astype(jnp.int32)
    typ = edge_type.astype(jnp.int32)
    sidx = _sidx(src, typ)

    h = _adapt(node_feature, nt, Wa, ba)

    for l in range(L):
        att_scaled = rel_att[l] * (rel_pri[l] / float(DK) ** 0.5)[:, :, None, None]
        wat = _blockdiag(att_scaled)
        wms = _blockdiag(rel_msg[l])
        tab, qp = _tables(
            h, nt, Qw[l], Qb[l], Kw[l], Kb[l], Vw[l], Vb[l], wat, wms)
        o0, o1 = _get_edge_kernel()(
            tab.reshape(R * N, 2 * D), qp, sidx, dst)
        alpha = jax.nn.sigmoid(skip[l]).reshape(1, T)
        h = _update(o0, o1, h, nt, Aw[l], Ab[l], alpha)

    return _head(h, Wc, bc.reshape(1, OUT))


# speedup vs baseline: 59.5671x; 1.2587x over previous
import functools

import jax
import jax.numpy as jnp
from jax import lax
from jax.experimental import pallas as pl
from jax.experimental.pallas import tpu as pltpu
from jax.experimental.pallas import tpu_sc as plsc

N = 50000
E = 800000
D_IN = 128
D = 64
T = 2
R = 4
H = 4
DK = D // H
L = 2
OUT = 2

NC = 2
NS = 16
LANES = 16

EPT = E // NS
EC = 48
NCHUNK = EPT // EC
ECT = EPT - NCHUNK * EC
ACC_W = 34
RSTR = 3200
ZB_FULL = 66
ZB_LAST = 41

_BLK = 2000
_GRID = N // _BLK


def _adapt_body(x_ref, nt_ref, wa_ref, ba_ref, h_ref):
    x = x_ref[...]
    y0 = jnp.dot(x, wa_ref[0], preferred_element_type=jnp.float32) + ba_ref[0, :][None, :]
    y1 = jnp.dot(x, wa_ref[1], preferred_element_type=jnp.float32) + ba_ref[1, :][None, :]
    sel = (nt_ref[...] == 0)
    h_ref[...] = jnp.tanh(jnp.where(sel, y0, y1))


def _adapt(x, nt, Wa, ba):
    return pl.pallas_call(
        _adapt_body,
        grid=(_GRID,),
        in_specs=[
            pl.BlockSpec((_BLK, D_IN), lambda i: (i, 0)),
            pl.BlockSpec((_BLK, 1), lambda i: (i, 0)),
            pl.BlockSpec((T, D_IN, D), lambda i: (0, 0, 0)),
            pl.BlockSpec((T, D), lambda i: (0, 0)),
        ],
        out_specs=pl.BlockSpec((_BLK, D), lambda i: (i, 0)),
        out_shape=jax.ShapeDtypeStruct((N, D), jnp.float32),
    )(x, nt, Wa, ba)


def _typed(h, nt0, w_ref, b_ref):
    y0 = jnp.dot(h, w_ref[0], preferred_element_type=jnp.float32) + b_ref[0, :][None, :]
    y1 = jnp.dot(h, w_ref[1], preferred_element_type=jnp.float32) + b_ref[1, :][None, :]
    return jnp.where(nt0, y0, y1)


def _tables_body(h_ref, nt_ref, qw_ref, qb_ref, kw_ref, kb_ref, vw_ref, vb_ref,
                 wat_ref, wms_ref, tab_ref, qp_ref):
    h = h_ref[...]
    nt0 = (nt_ref[...] == 0)
    q = _typed(h, nt0, qw_ref, qb_ref)
    k = _typed(h, nt0, kw_ref, kb_ref)
    v = _typed(h, nt0, vw_ref, vb_ref)
    qp_ref[...] = jnp.concatenate([q, jnp.zeros_like(q)], axis=1)
    for r in range(R):
        kt = jnp.dot(k, wat_ref[r], preferred_element_type=jnp.float32)
        mt = jnp.dot(v, wms_ref[r], preferred_element_type=jnp.float32)
        tab_ref[r] = jnp.concatenate([kt, mt], axis=1)


def _tables(h, nt, qw, qb, kw, kb, vw, vb, wat, wms):
    wspec = lambda shp: pl.BlockSpec(shp, lambda i: tuple(0 for _ in shp))
    return pl.pallas_call(
        _tables_body,
        grid=(_GRID,),
        in_specs=[
            pl.BlockSpec((_BLK, D), lambda i: (i, 0)),
            pl.BlockSpec((_BLK, 1), lambda i: (i, 0)),
            wspec((T, D, D)), wspec((T, D)),
            wspec((T, D, D)), wspec((T, D)),
            wspec((T, D, D)), wspec((T, D)),
            wspec((R, D, D)), wspec((R, D, D)),
        ],
        out_specs=[
            pl.BlockSpec((R, _BLK, 2 * D), lambda i: (0, i, 0)),
            pl.BlockSpec((_BLK, 2 * D), lambda i: (i, 0)),
        ],
        out_shape=[
            jax.ShapeDtypeStruct((R, N, 2 * D), jnp.float32),
            jax.ShapeDtypeStruct((N, 2 * D), jnp.float32),
        ],
    )(h, nt, qw, qb, kw, kb, vw, vb, wat, wms)


def _update_body(o0_ref, o1_ref, h_ref, nt_ref, aw_ref, ab_ref, al_ref, ho_ref):
    eps = 1e-16
    o0 = o0_ref[...]
    o1 = o1_ref[...]
    parts = []
    for src, base in ((o0, 0), (o1, 0)):
        for hh in range(2):
            den = src[:, 32 + hh:33 + hh]
            parts.append(src[:, hh * 16:(hh + 1) * 16] / (den + eps))
    agg = jnp.concatenate(parts, axis=1)
    g = agg * 0.5 * (1.0 + lax.erf(agg * (2.0 ** -0.5)))
    nt0 = (nt_ref[...] == 0)
    trans = _typed(g, nt0, aw_ref, ab_ref)
    alpha = jnp.where(nt0, al_ref[0, 0], al_ref[0, 1])
    ho_ref[...] = alpha * trans + (1.0 - alpha) * h_ref[...]


def _update(o0, o1, h, nt, aw, ab, alpha):
    return pl.pallas_call(
        _update_body,
        grid=(_GRID,),
        in_specs=[
            pl.BlockSpec((_BLK, ACC_W), lambda i: (i, 0)),
            pl.BlockSpec((_BLK, ACC_W), lambda i: (i, 0)),
            pl.BlockSpec((_BLK, D), lambda i: (i, 0)),
            pl.BlockSpec((_BLK, 1), lambda i: (i, 0)),
            pl.BlockSpec((T, D, D), lambda i: (0, 0, 0)),
            pl.BlockSpec((T, D), lambda i: (0, 0)),
            pl.BlockSpec((1, T), lambda i: (0, 0)),
        ],
        out_specs=pl.BlockSpec((_BLK, D), lambda i: (i, 0)),
        out_shape=jax.ShapeDtypeStruct((N, D), jnp.float32),
    )(o0, o1, h, nt, aw, ab, alpha)


def _head_body(h_ref, wc_ref, bc_ref, out_ref):
    out_ref[...] = (jnp.dot(h_ref[...], wc_ref[...],
                            preferred_element_type=jnp.float32)
                    + bc_ref[0, :][None, :])


def _head(h, Wc, bc):
    return pl.pallas_call(
        _head_body,
        grid=(_GRID,),
        in_specs=[
            pl.BlockSpec((_BLK, D), lambda i: (i, 0)),
            pl.BlockSpec((D, OUT), lambda i: (0, 0)),
            pl.BlockSpec((1, OUT), lambda i: (0, 0)),
        ],
        out_specs=pl.BlockSpec((_BLK, OUT), lambda i: (i, 0)),
        out_shape=jax.ShapeDtypeStruct((N, OUT), jnp.float32),
    )(h, Wc, bc)


def _edge_chunks(tab, qtab, sidx_hbm, dst_hbm, sid, acc, srcb, dstb,
                 srcb2, dstb2, srct, dstt, tabb, qb, msgb,
                 sem_t, sem_q, sem_a, sem_b, ko, mo, qo):
    iota = lax.iota(jnp.int32, LANES)
    zero16 = jnp.zeros((LANES,), jnp.float32)
    perms = [jnp.bitwise_xor(iota, sh) for sh in (8, 4, 2, 1)]

    def allsum(v):
        for p in perms:
            v = v + v[p]
        return v

    ebase = sid * EPT

    def compute_scatter(n, sb, db):
        cp_t = pltpu.async_copy(tab.at[sb], tabb.at[pl.ds(0, n)], sem_t)
        cp_q = pltpu.async_copy(qtab.at[db], qb.at[pl.ds(0, n)], sem_q)
        return cp_t, cp_q

    def run_edges(n, db):
        def edge(e, _):
            q0 = qb[e, pl.ds(qo, 16)]
            q1 = qb[e, pl.ds(qo + 16, 16)]
            k0 = tabb[e, pl.ds(ko, 16)]
            k1 = tabb[e, pl.ds(ko + 16, 16)]
            m0 = tabb[e, pl.ds(mo, 16)]
            m1 = tabb[e, pl.ds(mo + 16, 16)]
            w0 = jnp.exp(allsum(k0 * q0))
            w1 = jnp.exp(allsum(k1 * q1))
            den = jnp.where(iota == 14, w0, jnp.where(iota == 15, w1, zero16))
            msgb[e, pl.ds(18, 16)] = den
            msgb[e, pl.ds(0, 16)] = m0 * w0
            msgb[e, pl.ds(16, 16)] = m1 * w1
            return 0
        lax.fori_loop(0, n, edge, 0)
        pltpu.sync_copy(msgb.at[pl.ds(0, n)], acc.at[db], add=True)

    def idx_start(base, sb, db, sem):
        c1 = pltpu.async_copy(sidx_hbm.at[pl.ds(base, EC)], sb, sem)
        c2 = pltpu.async_copy(dst_hbm.at[pl.ds(base, EC)], db, sem)
        return c1, c2

    def idx_wait(sb, db, sem):
        pltpu.make_async_copy(sidx_hbm.at[pl.ds(0, EC)], sb, sem).wait()
        pltpu.make_async_copy(dst_hbm.at[pl.ds(0, EC)], db, sem).wait()

    def do_chunk(base, n, sb, db):
        ci_s = pltpu.async_copy(sidx_hbm.at[pl.ds(base, n)], sb, sem_t)
        ci_d = pltpu.async_copy(dst_hbm.at[pl.ds(base, n)], db, sem_q)
        ci_s.wait()
        ci_d.wait()
        cp_t, cp_q = compute_scatter(n, sb, db)
        cp_t.wait()
        cp_q.wait()
        run_edges(n, db)

    NPAIR = NCHUNK // 2
    idx_start(ebase, srcb, dstb, sem_a)

    def pair(i, _):
        idx_wait(srcb, dstb, sem_a)
        ga_t, ga_q = compute_scatter(EC, srcb, dstb)
        idx_start(ebase + (2 * i + 1) * EC, srcb2, dstb2, sem_b)
        ga_t.wait()
        ga_q.wait()
        run_edges(EC, dstb)
        idx_wait(srcb2, dstb2, sem_b)
        gb_t, gb_q = compute_scatter(EC, srcb2, dstb2)
        nbase = jnp.minimum(ebase + (2 * i + 2) * EC, E - EC)
        idx_start(nbase, srcb, dstb, sem_a)
        gb_t.wait()
        gb_q.wait()
        run_edges(EC, dstb2)
        return 0

    lax.fori_loop(0, NPAIR, pair, 0)
    idx_wait(srcb, dstb, sem_a)
    do_chunk(ebase + 2 * NPAIR * EC, EC, srcb, dstb)
    do_chunk(ebase + NCHUNK * EC, ECT, srct, dstt)


def _edge_kernel_body(tab, qp, sidx, dst, out0, out1,
                      acc, srcb, dstb, srcb2, dstb2, srct, dstt,
                      tabb, qb, msgb, sem_t, sem_q, sem_a, sem_b):
    cid = lax.axis_index("c")
    sid = lax.axis_index("s")
    zero16 = jnp.zeros((LANES,), jnp.float32)
    iota = lax.iota(jnp.int32, LANES)

    def fill_idx(buf, nwords, base):
        def st(g, _):
            buf[pl.ds(g * 16, 16)] = iota + (base + g * 16)
            return 0
        lax.fori_loop(0, nwords // 16, st, 0)

    def zrow(e, _):
        msgb[e, pl.ds(0, 16)] = zero16
        msgb[e, pl.ds(16, 16)] = zero16
        msgb[e, pl.ds(18, 16)] = zero16
        return 0
    lax.fori_loop(0, EC, zrow, 0)
    zbase = sid * RSTR
    nzb = jnp.where(sid == NS - 1, ZB_LAST, ZB_FULL)

    def zcopy(zb, _):
        fill_idx(srcb, EC, zbase + zb * EC)
        pltpu.sync_copy(msgb, acc.at[srcb])
        return 0
    lax.fori_loop(0, nzb, zcopy, 0)
    fill_idx(srct, ECT, zbase + nzb * EC)
    pltpu.sync_copy(msgb.at[pl.ds(0, ECT)], acc.at[srct])
    plsc.subcore_barrier()

    @pl.when(cid == 0)
    def _():
        _edge_chunks(tab, qp, sidx, dst, sid, acc, srcb, dstb, srcb2, dstb2,
                     srct, dstt, tabb, qb, msgb, sem_t, sem_q, sem_a, sem_b,
                     0, 64, 0)

    @pl.when(cid == 1)
    def _():
        _edge_chunks(tab, qp, sidx, dst, sid, acc, srcb, dstb, srcb2, dstb2,
                     srct, dstt, tabb, qb, msgb, sem_t, sem_q, sem_a, sem_b,
                     32, 96, 32)

    plsc.subcore_barrier()

    def wcopy_to(out):
        def wcopy(zb, _):
            fill_idx(srcb, EC, zbase + zb * EC)
            pltpu.sync_copy(acc.at[srcb], msgb)
            pltpu.sync_copy(msgb, out.at[pl.ds(zbase + zb * EC, EC)])
            return 0
        return wcopy

    def wtail_to(out):
        fill_idx(srct, ECT, zbase + nzb * EC)
        pltpu.sync_copy(acc.at[srct], msgb.at[pl.ds(0, ECT)])
        pltpu.sync_copy(msgb.at[pl.ds(0, ECT)],
                        out.at[pl.ds(zbase + nzb * EC, ECT)])

    @pl.when(cid == 0)
    def _():
        lax.fori_loop(0, nzb, wcopy_to(out0), 0)
        wtail_to(out0)

    @pl.when(cid == 1)
    def _():
        lax.fori_loop(0, nzb, wcopy_to(out1), 0)
        wtail_to(out1)


@functools.lru_cache(maxsize=1)
def _get_edge_kernel():
    return pl.kernel(
        _edge_kernel_body,
        out_type=[
            jax.ShapeDtypeStruct((N, ACC_W), jnp.float32),
            jax.ShapeDtypeStruct((N, ACC_W), jnp.float32),
        ],
        mesh=plsc.VectorSubcoreMesh(core_axis_name="c", subcore_axis_name="s",
                                    num_cores=NC, num_subcores=NS),
        scratch_types=[
            pltpu.VMEM_SHARED((N, ACC_W), jnp.float32),
            pltpu.VMEM((EC,), jnp.int32),
            pltpu.VMEM((EC,), jnp.int32),
            pltpu.VMEM((EC,), jnp.int32),
            pltpu.VMEM((EC,), jnp.int32),
            pltpu.VMEM((ECT,), jnp.int32),
            pltpu.VMEM((ECT,), jnp.int32),
            pltpu.VMEM((EC, 2 * D), jnp.float32),
            pltpu.VMEM((EC, 2 * D), jnp.float32),
            pltpu.VMEM((EC, ACC_W), jnp.float32),
            pltpu.SemaphoreType.DMA,
            pltpu.SemaphoreType.DMA,
            pltpu.SemaphoreType.DMA,
            pltpu.SemaphoreType.DMA,
        ],
    )


def _sidx_body(src_ref, typ_ref, out_ref):
    out_ref[...] = typ_ref[...] * N + src_ref[...]


def _sidx(src, typ):
    e2 = E // 128
    blk = e2
    return pl.pallas_call(
        _sidx_body,
        grid=(e2 // blk,),
        in_specs=[
            pl.BlockSpec((blk, 128), lambda i: (i, 0)),
            pl.BlockSpec((blk, 128), lambda i: (i, 0)),
        ],
        out_specs=pl.BlockSpec((blk, 128), lambda i: (i, 0)),
        out_shape=jax.ShapeDtypeStruct((e2, 128), jnp.int32),
    )(src.reshape(e2, 128), typ.reshape(e2, 128)).reshape(E)


def _blockdiag(mats):
    r, h, dk, _ = mats.shape
    eye = jnp.eye(h, dtype=mats.dtype)
    big = jnp.einsum('rhij,hg->rhigj', mats, eye).reshape(r, h, dk, h * dk)
    return big.reshape(r, h * dk, h * dk)


def kernel(node_feature, node_type, edge_time, edge_index, edge_type,
           Wa, ba, Qw, Qb, Kw, Kb, Vw, Vb, Aw, Ab,
           rel_pri, rel_att, rel_msg, skip, Wc, bc):
    del edge_time
    nt = node_type.astype(jnp.int32).reshape(N, 1)
    src = edge_index[0].astype(jnp.int32)
    dst = edge_index[1].astype(jnp.int32)
    typ = edge_type.astype(jnp.int32)
    sidx = _sidx(src, typ)

    h = _adapt(node_feature, nt, Wa, ba)

    for l in range(L):
        att_scaled = rel_att[l] * (rel_pri[l] / float(DK) ** 0.5)[:, :, None, None]
        wat = _blockdiag(att_scaled)
        wms = _blockdiag(rel_msg[l])
        tab, qp = _tables(
            h, nt, Qw[l], Qb[l], Kw[l], Kb[l], Vw[l], Vb[l], wat, wms)
        o0, o1 = _get_edge_kernel()(
            tab.reshape(R * N, 2 * D), qp, sidx, dst)
        alpha = jax.nn.sigmoid(skip[l]).reshape(1, T)
        h = _update(o0, o1, h, nt, Aw[l], Ab[l], alpha)

    return _head(h, Wc, bc.reshape(1, OUT))
